# baseline (speedup 1.0000x reference)
import jax
import jax.numpy as jnp
from jax import lax
from jax.experimental import pallas as pl
from jax.experimental.pallas import tpu as pltpu
from jax.experimental.pallas import tpu_sc as plsc

B = 16384
D = 64
NC = 2
NS = 16
NW = NC * NS
PER_W = B // NW
HALF = PER_W // 4
CHUNK = 128
NCHUNK = PER_W // CHUNK
IDX_ROWS = B // CHUNK
PAIR_ROWS = 100000 // 2


def _sc_body(u2d, i2d, j2d, t0, t1, ti, part_hbm, reg_hbm,
             uidx, pidx, nidx, uq, pq, nq, upar, ppar, npar,
             u_v, p_v, n_v, part_v, reg_v, sem_u, sem_pn):
    c = lax.axis_index("c")
    s = lax.axis_index("s")
    wid = s * NC + c
    rbase = wid * NCHUNK

    pltpu.sync_copy(u2d.at[pl.ds(rbase, NCHUNK)], uidx)
    pltpu.sync_copy(i2d.at[pl.ds(rbase, NCHUNK)], pidx)
    pltpu.sync_copy(j2d.at[pl.ds(rbase, NCHUNK)], nidx)

    def idx_body(t, carry):
        k = t >> 3
        off = (t & 7) * 16
        for src, qdst, pardst in ((uidx, uq, upar), (pidx, pq, ppar),
                                  (nidx, nq, npar)):
            v = src[k, pl.ds(off, 16)]
            qdst[k, pl.ds(off, 16)] = jnp.right_shift(v, 1)
            pardst[pl.ds(t * 16, 16)] = jnp.left_shift(
                jnp.bitwise_and(v, 1), 6)
        return carry

    lax.fori_loop(0, PER_W // 16, idx_body, 0)

    reg_acc = jnp.zeros((16,), jnp.float32)
    for h in range(4):
        ebase = h * HALF
        u0_desc = pltpu.async_copy(t0.at[uq.at[h]], u_v, sem_u)
        p_desc = pltpu.async_copy(ti.at[pq.at[h]], p_v, sem_pn)
        n_desc = pltpu.async_copy(ti.at[nq.at[h]], n_v, sem_pn)
        u0_desc.wait()
        u1_desc = pltpu.async_copy(t1.at[uq.at[h]], u_v, sem_u, add=True)
        u1_desc.wait()
        p_desc.wait()
        n_desc.wait()

        def body(e, reg_acc):
            bu = upar[pl.ds(ebase + e, 16)][0]
            bp = ppar[pl.ds(ebase + e, 16)][0]
            bn = npar[pl.ds(ebase + e, 16)][0]
            acc = jnp.zeros((16,), jnp.float32)
            for cc in range(D // 16):
                uc = u_v[e, pl.ds(bu + 16 * cc, 16)]
                pc = p_v[e, pl.ds(bp + 16 * cc, 16)]
                nc = n_v[e, pl.ds(bn + 16 * cc, 16)]
                acc = acc + uc * (nc - pc)
                reg_acc = reg_acc + uc * uc
            ee = ebase + e
            part_v[ee >> 3, pl.ds((ee & 7) * 16, 16)] = acc
            return reg_acc

        reg_acc = lax.fori_loop(0, HALF, body, reg_acc)

    reg_v[...] = reg_acc
    pltpu.sync_copy(part_v, part_hbm.at[pl.ds(wid * (PER_W // 8), PER_W // 8)])
    pltpu.sync_copy(reg_v, reg_hbm.at[wid])


_sc_kernel = pl.kernel(
    _sc_body,
    out_type=(jax.ShapeDtypeStruct((B // 8, 128), jnp.float32),
              jax.ShapeDtypeStruct((NW, 16), jnp.float32)),
    mesh=plsc.VectorSubcoreMesh(core_axis_name="c", subcore_axis_name="s",
                                num_cores=NC, num_subcores=NS),
    scratch_types=[
        pltpu.VMEM((NCHUNK, CHUNK), jnp.int32),
        pltpu.VMEM((NCHUNK, CHUNK), jnp.int32),
        pltpu.VMEM((NCHUNK, CHUNK), jnp.int32),
        pltpu.VMEM((NCHUNK, CHUNK), jnp.int32),
        pltpu.VMEM((NCHUNK, CHUNK), jnp.int32),
        pltpu.VMEM((NCHUNK, CHUNK), jnp.int32),
        pltpu.VMEM((PER_W + 16,), jnp.int32),
        pltpu.VMEM((PER_W + 16,), jnp.int32),
        pltpu.VMEM((PER_W + 16,), jnp.int32),
        pltpu.VMEM((HALF, 2 * D), jnp.float32),
        pltpu.VMEM((HALF, 2 * D), jnp.float32),
        pltpu.VMEM((HALF, 2 * D), jnp.float32),
        pltpu.VMEM((PER_W // 8, 128), jnp.float32),
        pltpu.VMEM((16,), jnp.float32),
        pltpu.SemaphoreType.DMA,
        pltpu.SemaphoreType.DMA,
    ],
    compiler_params=pltpu.CompilerParams(use_tc_tiling_on_sc=True),
)


def _ep_body(part_ref, regp_ref, loss_ref, reg_ref):
    lane = lax.broadcasted_iota(jnp.int32, (128, 8), 0)
    grp = lax.broadcasted_iota(jnp.int32, (128, 8), 1)
    sel = (lane // 16 == grp).astype(jnp.float32)
    x = jnp.dot(part_ref[...], sel,
                preferred_element_type=jnp.float32) * 0.5
    sp = jnp.maximum(x, 0.0) + jnp.log(1.0 + jnp.exp(-jnp.abs(x)))
    loss_ref[...] = jnp.sum(sp, keepdims=True) * (1.0 / B)
    reg_ref[...] = jnp.sum(regp_ref[...], keepdims=True) * (0.125 / B)


_ep_kernel = pl.pallas_call(
    _ep_body,
    out_shape=(jax.ShapeDtypeStruct((1, 1), jnp.float32),
               jax.ShapeDtypeStruct((1, 1), jnp.float32)),
)


def kernel(users, item_i, item_j, user_table0, user_table1, item_table):
    u2d = users.astype(jnp.int32).reshape(IDX_ROWS, CHUNK)
    i2d = item_i.astype(jnp.int32).reshape(IDX_ROWS, CHUNK)
    j2d = item_j.astype(jnp.int32).reshape(IDX_ROWS, CHUNK)
    t0p = user_table0.reshape(PAIR_ROWS, 2 * D)
    t1p = user_table1.reshape(PAIR_ROWS, 2 * D)
    tip = item_table.reshape(PAIR_ROWS, 2 * D)
    part_raw, reg_raw = _sc_kernel(u2d, i2d, j2d, t0p, t1p, tip)
    loss2d, reg2d = _ep_kernel(part_raw, reg_raw)
    return (loss2d[0, 0], reg2d[0, 0])


# speedup vs baseline: 1.4514x; 1.4514x over previous
import jax
import jax.numpy as jnp
from jax import lax
from jax.experimental import pallas as pl
from jax.experimental.pallas import tpu as pltpu
from jax.experimental.pallas import tpu_sc as plsc

B = 16384
D = 64
NC = 2
NS = 16
NW = NC * NS
PER_W = B // NW
PASS = 128
NPASS = PER_W // PASS
CHUNK = 128
NCHUNK = PER_W // CHUNK
IDX_ROWS = B // CHUNK


def _sc_body(u2d, i2d, j2d, t0, t1, ti, part_hbm, reg_hbm,
             uidx, pidx, nidx, u0_v, u1_v, p_v, n_v, part_v, reg_v,
             sem_u, sem_pn):
    c = lax.axis_index("c")
    s = lax.axis_index("s")
    wid = s * NC + c
    rbase = wid * NCHUNK

    pltpu.sync_copy(u2d.at[pl.ds(rbase, NCHUNK)], uidx)
    pltpu.sync_copy(i2d.at[pl.ds(rbase, NCHUNK)], pidx)
    pltpu.sync_copy(j2d.at[pl.ds(rbase, NCHUNK)], nidx)

    reg_acc = jnp.zeros((16,), jnp.float32)
    for p in range(NPASS):

        @pl.loop(0, PASS // 16)
        def _issue(g):
            t = p * (PASS // 16) + g
            k = t >> 3
            off = (t & 7) * 16
            uvec = uidx[k, pl.ds(off, 16)]
            pvec = pidx[k, pl.ds(off, 16)]
            nvec = nidx[k, pl.ds(off, 16)]
            eb = g * 16
            for j in range(16):
                pltpu.async_copy(t0.at[uvec[j]], u0_v.at[eb + j], sem_u)
                pltpu.async_copy(t1.at[uvec[j]], u1_v.at[eb + j], sem_u)
                pltpu.async_copy(ti.at[pvec[j]], p_v.at[eb + j], sem_pn)
                pltpu.async_copy(ti.at[nvec[j]], n_v.at[eb + j], sem_pn)

        pltpu.make_async_copy(t0.at[pl.ds(0, PASS)], u0_v, sem_u).wait()
        pltpu.make_async_copy(t1.at[pl.ds(0, PASS)], u1_v, sem_u).wait()
        pltpu.make_async_copy(ti.at[pl.ds(0, PASS)], p_v, sem_pn).wait()
        pltpu.make_async_copy(ti.at[pl.ds(0, PASS)], n_v, sem_pn).wait()

        def body(e, reg_acc):
            acc = jnp.zeros((16,), jnp.float32)
            for cc in range(D // 16):
                sl = pl.ds(16 * cc, 16)
                uc = u0_v[e, sl] + u1_v[e, sl]
                pc = p_v[e, sl]
                nc = n_v[e, sl]
                acc = acc + uc * (nc - pc)
                reg_acc = reg_acc + uc * uc
            ee = p * PASS + e
            part_v[ee >> 3, pl.ds((ee & 7) * 16, 16)] = acc
            return reg_acc

        reg_acc = lax.fori_loop(0, PASS, body, reg_acc)

    reg_v[...] = reg_acc
    pltpu.sync_copy(part_v, part_hbm.at[pl.ds(wid * (PER_W // 8), PER_W // 8)])
    pltpu.sync_copy(reg_v, reg_hbm.at[wid])


_sc_kernel = pl.kernel(
    _sc_body,
    out_type=(jax.ShapeDtypeStruct((B // 8, 128), jnp.float32),
              jax.ShapeDtypeStruct((NW, 16), jnp.float32)),
    mesh=plsc.VectorSubcoreMesh(core_axis_name="c", subcore_axis_name="s",
                                num_cores=NC, num_subcores=NS),
    scratch_types=[
        pltpu.VMEM((NCHUNK, CHUNK), jnp.int32),
        pltpu.VMEM((NCHUNK, CHUNK), jnp.int32),
        pltpu.VMEM((NCHUNK, CHUNK), jnp.int32),
        pltpu.VMEM((PASS, D), jnp.float32),
        pltpu.VMEM((PASS, D), jnp.float32),
        pltpu.VMEM((PASS, D), jnp.float32),
        pltpu.VMEM((PASS, D), jnp.float32),
        pltpu.VMEM((PER_W // 8, 128), jnp.float32),
        pltpu.VMEM((16,), jnp.float32),
        pltpu.SemaphoreType.DMA,
        pltpu.SemaphoreType.DMA,
    ],
    compiler_params=pltpu.CompilerParams(use_tc_tiling_on_sc=True),
)


def _ep_body(part_ref, regp_ref, loss_ref, reg_ref):
    lane = lax.broadcasted_iota(jnp.int32, (128, 8), 0)
    grp = lax.broadcasted_iota(jnp.int32, (128, 8), 1)
    sel = (lane // 16 == grp).astype(jnp.float32)
    x = jnp.dot(part_ref[...], sel,
                preferred_element_type=jnp.float32) * 0.5
    sp = jnp.maximum(x, 0.0) + jnp.log(1.0 + jnp.exp(-jnp.abs(x)))
    loss_ref[...] = jnp.sum(sp, keepdims=True) * (1.0 / B)
    reg_ref[...] = jnp.sum(regp_ref[...], keepdims=True) * (0.125 / B)


_ep_kernel = pl.pallas_call(
    _ep_body,
    out_shape=(jax.ShapeDtypeStruct((1, 1), jnp.float32),
               jax.ShapeDtypeStruct((1, 1), jnp.float32)),
)


def kernel(users, item_i, item_j, user_table0, user_table1, item_table):
    u2d = users.astype(jnp.int32).reshape(IDX_ROWS, CHUNK)
    i2d = item_i.astype(jnp.int32).reshape(IDX_ROWS, CHUNK)
    j2d = item_j.astype(jnp.int32).reshape(IDX_ROWS, CHUNK)
    part_raw, reg_raw = _sc_kernel(u2d, i2d, j2d,
                                   user_table0, user_table1, item_table)
    loss2d, reg2d = _ep_kernel(part_raw, reg_raw)
    return (loss2d[0, 0], reg2d[0, 0])


# speedup vs baseline: 1.5276x; 1.0525x over previous
import jax
import jax.numpy as jnp
from jax import lax
from jax.experimental import pallas as pl
from jax.experimental.pallas import tpu as pltpu
from jax.experimental.pallas import tpu_sc as plsc

B = 16384
D = 64
V = 100000
NC = 2
NS = 16
NW = NC * NS
DPW = D // NW
IC = 4096
GB = 16


def _sc_body(users, item_i, item_j, t0, t1, ti, uf_hbm, pn_hbm,
             row_v, buf_v, ia_v, ib_v):
    c = lax.axis_index("c")
    s = lax.axis_index("s")
    w = s * NC + c

    for dd in range(DPW):
        d = w * DPW + dd

        pltpu.sync_copy(ti.at[d], row_v)

        @pl.loop(0, B // IC)
        def _pn_chunks(ck):
            pltpu.sync_copy(item_i.at[pl.ds(ck * IC, IC)], ia_v)
            pltpu.sync_copy(item_j.at[pl.ds(ck * IC, IC)], ib_v)

            @pl.loop(0, IC // GB)
            def _pn_g(g):
                iv = ia_v[pl.ds(g * GB, GB)]
                jv = ib_v[pl.ds(g * GB, GB)]
                pos = plsc.load_gather(row_v, [iv])
                neg = plsc.load_gather(row_v, [jv])
                buf_v[pl.ds(ck * IC + g * GB, GB)] = neg - pos

        pltpu.sync_copy(buf_v, pn_hbm.at[d])

        pltpu.sync_copy(t0.at[d], row_v)

        @pl.loop(0, B // IC)
        def _u0_chunks(ck):
            pltpu.sync_copy(users.at[pl.ds(ck * IC, IC)], ia_v)

            @pl.loop(0, IC // GB)
            def _u0_g(g):
                uv = ia_v[pl.ds(g * GB, GB)]
                buf_v[pl.ds(ck * IC + g * GB, GB)] = plsc.load_gather(row_v, [uv])

        pltpu.sync_copy(t1.at[d], row_v)

        @pl.loop(0, B // IC)
        def _u1_chunks(ck):
            pltpu.sync_copy(users.at[pl.ds(ck * IC, IC)], ia_v)

            @pl.loop(0, IC // GB)
            def _u1_g(g):
                uv = ia_v[pl.ds(g * GB, GB)]
                o = ck * IC + g * GB
                buf_v[pl.ds(o, GB)] = (buf_v[pl.ds(o, GB)]
                                       + plsc.load_gather(row_v, [uv]))

        pltpu.sync_copy(buf_v, uf_hbm.at[d])


_sc_kernel = pl.kernel(
    _sc_body,
    out_type=(jax.ShapeDtypeStruct((D, B), jnp.float32),
              jax.ShapeDtypeStruct((D, B), jnp.float32)),
    mesh=plsc.VectorSubcoreMesh(core_axis_name="c", subcore_axis_name="s",
                                num_cores=NC, num_subcores=NS),
    scratch_types=[
        pltpu.VMEM((V,), jnp.float32),
        pltpu.VMEM((B,), jnp.float32),
        pltpu.VMEM((IC,), jnp.int32),
        pltpu.VMEM((IC,), jnp.int32),
    ],
    compiler_params=pltpu.CompilerParams(use_tc_tiling_on_sc=True,
                                         needs_layout_passes=False),
)


def _ep_body(uf_ref, pn_ref, loss_ref, reg_ref):
    uf = uf_ref[...]
    pn = pn_ref[...]
    x = jnp.sum(uf * pn, axis=0, keepdims=True) * 0.5
    sp = jnp.maximum(x, 0.0) + jnp.log(1.0 + jnp.exp(-jnp.abs(x)))
    loss_ref[...] = jnp.sum(sp, keepdims=True) * (1.0 / B)
    reg_ref[...] = jnp.sum(uf * uf, keepdims=True) * (0.125 / B)


_ep_kernel = pl.pallas_call(
    _ep_body,
    out_shape=(jax.ShapeDtypeStruct((1, 1), jnp.float32),
               jax.ShapeDtypeStruct((1, 1), jnp.float32)),
)


def kernel(users, item_i, item_j, user_table0, user_table1, item_table):
    u1d = users.astype(jnp.int32)
    i1d = item_i.astype(jnp.int32)
    j1d = item_j.astype(jnp.int32)
    uf, pn = _sc_kernel(u1d, i1d, j1d,
                        user_table0.T, user_table1.T, item_table.T)
    loss2d, reg2d = _ep_kernel(uf, pn)
    return (loss2d[0, 0], reg2d[0, 0])
